# baseline (speedup 1.0000x reference)
import functools

import jax
import jax.numpy as jnp
from jax import lax
from jax.experimental import pallas as pl
from jax.experimental.pallas import tpu as pltpu
from jax.experimental.pallas import tpu_sc as plsc

_L = 16


@functools.lru_cache(maxsize=None)
def _make_sc_permute(B, C, H, W):
    info = plsc.get_sparse_core_info()
    NC, NS = info.num_cores, info.num_subcores
    NW = NC * NS
    assert C % _L == 0 and B % NW == 0
    bpt = B // NW
    n_chunks = C // _L
    mesh = plsc.VectorSubcoreMesh(core_axis_name="c", subcore_axis_name="s")

    @functools.partial(
        pl.kernel,
        mesh=mesh,
        compiler_params=pltpu.CompilerParams(needs_layout_passes=False),
        out_type=jax.ShapeDtypeStruct((B * C, H, W), jnp.float32),
        scratch_types=[
            pltpu.VMEM((C,), jnp.float32),
            pltpu.VMEM((C,), jnp.int32),
            pltpu.SemaphoreType.DMA,
        ],
    )
    def k(indexes_hbm, in_hbm, out_hbm, idxs_v, sel_v, dsem):
        wid = lax.axis_index("s") * NC + lax.axis_index("c")
        pltpu.sync_copy(indexes_hbm, idxs_v)

        iota = lax.iota(jnp.int32, _L)
        one = jnp.int32(1)
        zero = jnp.int32(0)

        acc = jnp.zeros((_L,), jnp.int32)
        for c in range(n_chunks):
            v = idxs_v[pl.ds(c * _L, _L)]
            acc = acc + jnp.where(v != 0.0, one, zero)
        total_nz = zero
        for j in range(_L):
            total_nz = total_nz + acc[j]

        nz = zero
        z = zero
        for c in range(n_chunks):
            v = idxs_v[pl.ds(c * _L, _L)]
            posvec = jnp.zeros((_L,), jnp.int32)
            for j in range(_L):
                mj = v[j] != 0.0
                pos_j = jnp.where(mj, nz, total_nz + z)
                posvec = jnp.where(iota == j, pos_j, posvec)
                nz = nz + jnp.where(mj, one, zero)
                z = z + jnp.where(mj, zero, one)
            plsc.store_scatter(sel_v, [posvec], iota + (c * _L))

        out_base = wid * bpt * C
        for b in range(bpt):
            row0 = (wid * bpt + b) * C
            for c in range(n_chunks):
                vec = sel_v[pl.ds(c * _L, _L)] + row0
                for j in range(_L):
                    pltpu.async_copy(
                        in_hbm.at[pl.ds(vec[j], 1)],
                        out_hbm.at[pl.ds(out_base + b * C + c * _L + j, 1)],
                        dsem,
                    )
        pltpu.make_async_copy(
            in_hbm.at[pl.ds(out_base, bpt * C)],
            out_hbm.at[pl.ds(out_base, bpt * C)],
            dsem,
        ).wait()

    return k


def kernel(input_tensor, indexes):
    B, C, H, W = input_tensor.shape
    flat = input_tensor.reshape(B * C, H, W)
    out = _make_sc_permute(B, C, H, W)(indexes, flat)
    return out.reshape(B, C, H, W)


# speedup vs baseline: 14.7704x; 14.7704x over previous
import functools

import jax
import jax.numpy as jnp
from jax import lax
from jax.experimental import pallas as pl
from jax.experimental.pallas import tpu as pltpu
from jax.experimental.pallas import tpu_sc as plsc

_L = 16
_G = 8


@functools.lru_cache(maxsize=None)
def _make_sc_permute(B, C, H, W):
    info = plsc.get_sparse_core_info()
    NC, NS = info.num_cores, info.num_subcores
    NW = NC * NS
    assert C % _L == 0 and B % NW == 0
    bpt = B // NW
    n_chunks = C // _L
    mesh = plsc.VectorSubcoreMesh(core_axis_name="c", subcore_axis_name="s")

    @functools.partial(
        pl.kernel,
        mesh=mesh,
        compiler_params=pltpu.CompilerParams(needs_layout_passes=False),
        out_type=jax.ShapeDtypeStruct((B * C, H, W), jnp.float32),
        scratch_types=[
            pltpu.VMEM((C,), jnp.float32),
            pltpu.VMEM((C,), jnp.int32),
            pltpu.VMEM((_G, H, W), jnp.float32),
            pltpu.VMEM((_G, H, W), jnp.float32),
            pltpu.SemaphoreType.DMA,
            pltpu.SemaphoreType.DMA,
            pltpu.SemaphoreType.DMA,
            pltpu.SemaphoreType.DMA,
        ],
    )
    def k(indexes_hbm, in_hbm, out_hbm, idxs_v, sel_v, buf0, buf1,
          g0, g1, p0, p1):
        wid = lax.axis_index("s") * NC + lax.axis_index("c")
        pltpu.sync_copy(indexes_hbm, idxs_v)

        iota = lax.iota(jnp.int32, _L)
        one = jnp.int32(1)
        zero = jnp.int32(0)

        acc = jnp.zeros((_L,), jnp.int32)
        for c in range(n_chunks):
            v = idxs_v[pl.ds(c * _L, _L)]
            acc = acc + jnp.where(v != 0.0, one, zero)
        total_nz = zero
        for j in range(_L):
            total_nz = total_nz + acc[j]

        nz = zero
        z = zero
        for c in range(n_chunks):
            v = idxs_v[pl.ds(c * _L, _L)]
            posvec = jnp.zeros((_L,), jnp.int32)
            for j in range(_L):
                mj = v[j] != 0.0
                pos_j = jnp.where(mj, nz, total_nz + z)
                posvec = jnp.where(iota == j, pos_j, posvec)
                nz = nz + jnp.where(mj, one, zero)
                z = z + jnp.where(mj, zero, one)
            plsc.store_scatter(sel_v, [posvec], iota + (c * _L))

        bufs = (buf0, buf1)
        gsems = (g0, g1)
        psems = (p0, p1)
        out_base = wid * bpt * C
        ngroups = bpt * C // _G
        gathers = [None] * ngroups
        writes = [None] * ngroups

        def start_gathers(g):
            b, off = divmod(g * _G, C)
            row0 = (wid * bpt + b) * C
            chunk, lane0 = divmod(off, _L)
            vec = sel_v[pl.ds(chunk * _L, _L)] + row0
            cps = []
            for j in range(_G):
                cps.append(pltpu.async_copy(
                    in_hbm.at[pl.ds(vec[lane0 + j], 1)],
                    bufs[g % 2].at[pl.ds(j, 1)],
                    gsems[g % 2],
                ))
            return cps

        def start_write(g):
            return pltpu.async_copy(
                bufs[g % 2],
                out_hbm.at[pl.ds(out_base + g * _G, _G)],
                psems[g % 2],
            )

        for g in range(ngroups):
            if g >= 2:
                writes[g - 2].wait()
            gathers[g] = start_gathers(g)
            if g >= 1:
                for cp in gathers[g - 1]:
                    cp.wait()
                writes[g - 1] = start_write(g - 1)
        for cp in gathers[ngroups - 1]:
            cp.wait()
        writes[ngroups - 1] = start_write(ngroups - 1)
        writes[ngroups - 2].wait()
        writes[ngroups - 1].wait()

    return k


def kernel(input_tensor, indexes):
    B, C, H, W = input_tensor.shape
    flat = input_tensor.reshape(B * C, H, W)
    out = _make_sc_permute(B, C, H, W)(indexes, flat)
    return out.reshape(B, C, H, W)
